# Initial kernel scaffold; baseline (speedup 1.0000x reference)
import jax
import jax.numpy as jnp
from jax.experimental import pallas as pl


def kernel(grid, matrix):
    raise NotImplementedError("write your pallas kernel here")


# speedup vs baseline: 1.4406x; 1.4406x over previous
import functools

import jax
import jax.numpy as jnp
from jax import lax
from jax.experimental import pallas as pl
from jax.experimental.pallas import tpu as pltpu
from jax.experimental.pallas import tpu_sc as plsc

C = 192
H = W = 384
GH = GW = 384
N = GH * GW
NPIX = H * W
NC, NS = 2, 16
NW = NC * NS
PTS_PER_W = N // NW
CHUNK = 64
NCHUNK = PTS_PER_W // CHUNK
CG = C // 16


def _sc_sample(table, idx4, w4):
    mesh = plsc.VectorSubcoreMesh(core_axis_name="c", subcore_axis_name="s")

    @functools.partial(
        pl.kernel,
        out_type=jax.ShapeDtypeStruct((N, C), jnp.float32),
        mesh=mesh,
        scratch_types=[
            pltpu.VMEM((4, CHUNK), jnp.int32),
            pltpu.VMEM((4, CHUNK), jnp.float32),
            pltpu.VMEM((4, CHUNK, C), jnp.float32),
            pltpu.VMEM((CHUNK, C), jnp.float32),
            pltpu.SemaphoreType.DMA,
        ],
        compiler_params=pltpu.CompilerParams(use_tc_tiling_on_sc=False),
    )
    def k(table_hbm, idx_hbm, w_hbm, out_hbm, idx_v, w_v, rows_v, out_v, sem):
        wid = lax.axis_index("s") * NC + lax.axis_index("c")

        def chunk_body(ci, carry):
            base = wid * PTS_PER_W + ci * CHUNK
            pltpu.sync_copy(idx_hbm.at[:, pl.ds(base, CHUNK)], idx_v)
            pltpu.sync_copy(w_hbm.at[:, pl.ds(base, CHUNK)], w_v)
            descs = [
                pltpu.async_copy(table_hbm.at[idx_v.at[j]], rows_v.at[j], sem)
                for j in range(4)
            ]
            for d in descs:
                d.wait()

            def blk_body(b, c2):
                wv = [w_v[j, pl.ds(b * 16, 16)] for j in range(4)]
                for k in range(16):
                    i = b * 16 + k
                    w00, w01, w10, w11 = wv[0][k], wv[1][k], wv[2][k], wv[3][k]
                    for g in range(CG):
                        s = pl.ds(g * 16, 16)
                        out_v[i, s] = (rows_v[0, i, s] * w00
                                       + rows_v[1, i, s] * w01
                                       + rows_v[2, i, s] * w10
                                       + rows_v[3, i, s] * w11)
                return c2

            lax.fori_loop(0, CHUNK // 16, blk_body, 0)
            pltpu.sync_copy(out_v, out_hbm.at[pl.ds(base, CHUNK)])
            return carry

        lax.fori_loop(0, NCHUNK, chunk_body, 0)

    return k(table, idx4, w4)


def kernel(grid, matrix):
    x = grid[0, :, :, 0].reshape(-1)
    y = grid[0, :, :, 1].reshape(-1)
    ix = jnp.clip(((x + 1.0) * W - 1.0) / 2.0, 0.0, W - 1.0)
    iy = jnp.clip(((y + 1.0) * H - 1.0) / 2.0, 0.0, H - 1.0)
    ix0f = jnp.floor(ix)
    iy0f = jnp.floor(iy)
    wx = ix - ix0f
    wy = iy - iy0f
    ix0 = jnp.clip(ix0f.astype(jnp.int32), 0, W - 1)
    ix1 = jnp.clip(ix0 + 1, 0, W - 1)
    iy0 = jnp.clip(iy0f.astype(jnp.int32), 0, H - 1)
    iy1 = jnp.clip(iy0 + 1, 0, H - 1)
    idx4 = jnp.stack([iy0 * W + ix0, iy0 * W + ix1,
                      iy1 * W + ix0, iy1 * W + ix1])
    w4 = jnp.stack([(1.0 - wy) * (1.0 - wx), (1.0 - wy) * wx,
                    wy * (1.0 - wx), wy * wx])
    table = matrix.reshape(C, NPIX).T
    out_flat = _sc_sample(table, idx4, w4)
    return out_flat.T.reshape(1, C, GH, GW)
